# baseline (speedup 1.0000x reference)
import functools

import jax
import jax.numpy as jnp
from jax import lax
from jax.experimental import pallas as pl
from jax.experimental.pallas import tpu as pltpu, tpu_sc as plsc

VOCAB = 1000000
DIM = 32
B = 16384
L = 50

NC = 2
NS = 16
NW = NC * NS

BAGS_PER_W = B // NW
CB = 64
RC = CB * L
NCHUNK = BAGS_PER_W // CB
GB = 128
NGB = RC // GB


def _mesh():
    return plsc.VectorSubcoreMesh(core_axis_name="c", subcore_axis_name="s")


TCH = 8192
TSUB = TCH // 4
TGRID = (VOCAB + TCH - 1) // TCH
VROWS = TGRID * TCH


def _tc_transpose_body(x_ref, o_ref):
    x = x_ref[...]
    eye = jnp.eye(DIM, dtype=jnp.float32)
    parts = [
        lax.dot_general(x[:, k * TSUB:(k + 1) * TSUB], eye,
                        (((0,), (0,)), ((), ())),
                        preferred_element_type=jnp.float32)
        for k in range(4)
    ]
    o_ref[...] = jnp.concatenate(parts, axis=1)


def _tc_transpose(tab_t):
    return pl.pallas_call(
        _tc_transpose_body,
        grid=(TGRID,),
        in_specs=[pl.BlockSpec((DIM, TCH), lambda i: (0, i))],
        out_specs=pl.BlockSpec((TSUB, 4 * DIM), lambda i: (i, 0)),
        out_shape=jax.ShapeDtypeStruct((TGRID * TSUB, 4 * DIM), jnp.float32),
    )(tab_t)


NFULL = VOCAB // 128
TAIL = VOCAB - NFULL * 128
ROUND = 4
NROUND = 62
BLKW = 128 * DIM


@functools.partial(
    pl.kernel,
    out_type=jax.ShapeDtypeStruct((VOCAB * DIM,), jnp.float32),
    mesh=_mesh(),
    compiler_params=pltpu.CompilerParams(
        needs_layout_passes=False, use_tc_tiling_on_sc=True),
    scratch_types=[
        pltpu.VMEM((2, ROUND, 4, 8, 129), jnp.float32),
        pltpu.VMEM((2, ROUND * BLKW), jnp.float32),
        pltpu.SemaphoreType.DMA,
        pltpu.SemaphoreType.DMA,
    ],
)
def _detile_kernel(tab_t, tail_rows, out_flat, buf, stage, sem_in, sem_out):
    wid = lax.axis_index("s") * NC + lax.axis_index("c")
    lane = lax.broadcasted_iota(jnp.int32, (16,), 0)
    dlo = lane // 8
    dhi = dlo + 2
    kv = lane % 8

    bstart = jnp.where(wid < 4, wid * 245, 980 + (wid - 4) * 244)
    bend = bstart + jnp.where(wid < 4, 245, 244)

    @pl.when(wid == NW - 1)
    def _copy_tail():
        cp = pltpu.make_async_copy(
            tail_rows, stage.at[0, pl.ds(0, TAIL * DIM)], sem_in)
        cp.start()
        cp.wait()
        cp = pltpu.make_async_copy(
            stage.at[0, pl.ds(0, TAIL * DIM)],
            out_flat.at[pl.ds(NFULL * BLKW, TAIL * DIM)], sem_out)
        cp.start()
        cp.wait()

    def in_copies(r, p):
        cps = []
        for bi in range(ROUND):
            cb = jnp.minimum(bstart + r * ROUND + bi, bend - 1)
            for rblk in range(4):
                cps.append(pltpu.make_async_copy(
                    tab_t.at[pl.ds(rblk * 8, 8), pl.ds(cb * 128, 128)],
                    buf.at[p, bi, rblk, :, pl.ds(0, 128)],
                    sem_in))
        return cps

    def round_body(r, carry):
        p = r % 2

        @pl.when(r + 1 < NROUND)
        def _fire_next():
            for cp in in_copies(r + 1, 1 - p):
                cp.start()

        for cp in in_copies(r, p):
            cp.wait()

        @pl.when(r >= 2)
        def _drain_outs():
            for bi in range(ROUND):
                pltpu.make_async_copy(
                    stage.at[p, pl.ds(bi * BLKW, BLKW)],
                    out_flat.at[pl.ds(bstart * BLKW + bi * BLKW, BLKW)],
                    sem_out).wait()

        psp = jnp.full((16,), p, jnp.int32)
        for bi in range(ROUND):
            bsp = jnp.full((16,), bi, jnp.int32)

            def row_body(v8, c, bi=bi, bsp=bsp):
                for u in range(8):
                    vi = v8 * 8 + u
                    vil = jnp.full((16,), vi, jnp.int32)
                    lo = plsc.load_gather(buf, [psp, bsp, dlo, kv, vil])
                    hi = plsc.load_gather(buf, [psp, bsp, dhi, kv, vil])
                    sbase = bi * BLKW + vi * DIM
                    stage[p, pl.ds(sbase, 16)] = lo
                    stage[p, pl.ds(sbase + 16, 16)] = hi
                return c

            lax.fori_loop(0, 16, row_body, 0)

        for bi in range(ROUND):
            cb = jnp.minimum(bstart + r * ROUND + bi, bend - 1)
            pltpu.make_async_copy(
                stage.at[p, pl.ds(bi * BLKW, BLKW)],
                out_flat.at[pl.ds(cb * BLKW, BLKW)],
                sem_out).start()
        return carry

    for cp in in_copies(0, 0):
        cp.start()
    lax.fori_loop(0, NROUND, round_body, 0)

    for _ in range(2):
        for bi in range(ROUND):
            pltpu.make_async_copy(
                stage.at[0, pl.ds(bi * BLKW, BLKW)],
                out_flat.at[pl.ds(bstart * BLKW + bi * BLKW, BLKW)],
                sem_out).wait()


@functools.partial(
    pl.kernel,
    out_type=jax.ShapeDtypeStruct((B, DIM), jnp.float32),
    mesh=_mesh(),
    compiler_params=pltpu.CompilerParams(
        needs_layout_passes=False, use_tc_tiling_on_sc=False),
    scratch_types=[
        pltpu.VMEM((RC,), jnp.int32),
        pltpu.VMEM((RC,), jnp.float32),
        pltpu.VMEM((RC, DIM), jnp.float32),
        pltpu.VMEM((CB, DIM), jnp.float32),
        pltpu.SemaphoreType.DMA,
        pltpu.SemaphoreType.DMA,
        pltpu.SemaphoreType.DMA,
    ],
)
def _bag_kernel(table_hbm, idx_hbm, w_hbm, out_hbm,
                idx_v, w_v, rows_v, out_v, sem_in, sem_rows, sem_out):
    wid = lax.axis_index("s") * NC + lax.axis_index("c")
    lane = lax.broadcasted_iota(jnp.int32, (16,), 0)

    def chunk_body(c, carry):
        bag_base = wid * BAGS_PER_W + c * CB
        row_base = bag_base * L

        cp_i = pltpu.make_async_copy(
            idx_hbm.at[pl.ds(row_base, RC)], idx_v, sem_in)
        cp_w = pltpu.make_async_copy(
            w_hbm.at[pl.ds(row_base, RC)], w_v, sem_in)
        cp_i.start()
        cp_w.start()
        cp_i.wait()
        cp_w.wait()

        def remap_body(i, carry):
            v = idx_v[pl.ds(i * 16, 16)]
            r = ((v >> 13) << 13) + ((v & 2047) << 2) + ((v >> 11) & 3)
            idx_v[pl.ds(i * 16, 16)] = r
            return carry

        lax.fori_loop(0, RC // 16, remap_body, 0)

        g = pltpu.make_async_copy(table_hbm.at[idx_v], rows_v, sem_rows)
        g.start()
        g.wait()

        def bag_body(b, carry):
            r0 = b * L
            acc = [jnp.zeros((16,), jnp.float32) for _ in range(4)]
            for j in range(L):
                r = r0 + j
                w16 = plsc.load_gather(w_v, [jnp.full((16,), r, jnp.int32)])
                lo = rows_v[r, pl.ds(0, 16)]
                hi = rows_v[r, pl.ds(16, 16)]
                k = 2 * (j % 2)
                acc[k] = acc[k] + w16 * lo
                acc[k + 1] = acc[k + 1] + w16 * hi
            out_v[b, pl.ds(0, 16)] = acc[0] + acc[2]
            out_v[b, pl.ds(16, 16)] = acc[1] + acc[3]
            return carry

        lax.fori_loop(0, CB, bag_body, 0)

        cp_o = pltpu.make_async_copy(
            out_v, out_hbm.at[pl.ds(bag_base, CB)], sem_out)
        cp_o.start()
        cp_o.wait()
        return carry

    lax.fori_loop(0, NCHUNK, chunk_body, 0)


def kernel(inputs, offsets, per_sample_weights, table):
    del offsets
    lin = _tc_transpose(table.T)
    return _bag_kernel(lin.reshape(VROWS, DIM), inputs, per_sample_weights)


# speedup vs baseline: 1.0202x; 1.0202x over previous
import functools

import jax
import jax.numpy as jnp
from jax import lax
from jax.experimental import pallas as pl
from jax.experimental.pallas import tpu as pltpu, tpu_sc as plsc

VOCAB = 1000000
DIM = 32
B = 16384
L = 50

NC = 2
NS = 16
NW = NC * NS

BAGS_PER_W = B // NW
CB = 64
RC = CB * L
NCHUNK = BAGS_PER_W // CB
GB = 128
NGB = RC // GB


def _mesh():
    return plsc.VectorSubcoreMesh(core_axis_name="c", subcore_axis_name="s")


TCH = 32768
TSUB = TCH // 4
SHB = 15
SHS = 13
TGRID = (VOCAB + TCH - 1) // TCH
VROWS = TGRID * TCH


def _tc_transpose_body(x_ref, o_ref):
    x = x_ref[...]
    parts = [jnp.transpose(x[:, k * TSUB:(k + 1) * TSUB], (1, 0))
             for k in range(4)]
    o_ref[...] = jnp.concatenate(parts, axis=1)


def _tc_transpose(tab_t):
    return pl.pallas_call(
        _tc_transpose_body,
        grid=(TGRID,),
        in_specs=[pl.BlockSpec((DIM, TCH), lambda i: (0, i))],
        out_specs=pl.BlockSpec((TSUB, 4 * DIM), lambda i: (i, 0)),
        out_shape=jax.ShapeDtypeStruct((TGRID * TSUB, 4 * DIM), jnp.float32),
    )(tab_t)


NFULL = VOCAB // 128
TAIL = VOCAB - NFULL * 128
ROUND = 4
NROUND = 62
BLKW = 128 * DIM


@functools.partial(
    pl.kernel,
    out_type=jax.ShapeDtypeStruct((VOCAB * DIM,), jnp.float32),
    mesh=_mesh(),
    compiler_params=pltpu.CompilerParams(
        needs_layout_passes=False, use_tc_tiling_on_sc=True),
    scratch_types=[
        pltpu.VMEM((2, ROUND, 4, 8, 129), jnp.float32),
        pltpu.VMEM((2, ROUND * BLKW), jnp.float32),
        pltpu.SemaphoreType.DMA,
        pltpu.SemaphoreType.DMA,
    ],
)
def _detile_kernel(tab_t, tail_rows, out_flat, buf, stage, sem_in, sem_out):
    wid = lax.axis_index("s") * NC + lax.axis_index("c")
    lane = lax.broadcasted_iota(jnp.int32, (16,), 0)
    dlo = lane // 8
    dhi = dlo + 2
    kv = lane % 8

    bstart = jnp.where(wid < 4, wid * 245, 980 + (wid - 4) * 244)
    bend = bstart + jnp.where(wid < 4, 245, 244)

    @pl.when(wid == NW - 1)
    def _copy_tail():
        cp = pltpu.make_async_copy(
            tail_rows, stage.at[0, pl.ds(0, TAIL * DIM)], sem_in)
        cp.start()
        cp.wait()
        cp = pltpu.make_async_copy(
            stage.at[0, pl.ds(0, TAIL * DIM)],
            out_flat.at[pl.ds(NFULL * BLKW, TAIL * DIM)], sem_out)
        cp.start()
        cp.wait()

    def in_copies(r, p):
        cps = []
        for bi in range(ROUND):
            cb = jnp.minimum(bstart + r * ROUND + bi, bend - 1)
            for rblk in range(4):
                cps.append(pltpu.make_async_copy(
                    tab_t.at[pl.ds(rblk * 8, 8), pl.ds(cb * 128, 128)],
                    buf.at[p, bi, rblk, :, pl.ds(0, 128)],
                    sem_in))
        return cps

    def round_body(r, carry):
        p = r % 2

        @pl.when(r + 1 < NROUND)
        def _fire_next():
            for cp in in_copies(r + 1, 1 - p):
                cp.start()

        for cp in in_copies(r, p):
            cp.wait()

        @pl.when(r >= 2)
        def _drain_outs():
            for bi in range(ROUND):
                pltpu.make_async_copy(
                    stage.at[p, pl.ds(bi * BLKW, BLKW)],
                    out_flat.at[pl.ds(bstart * BLKW + bi * BLKW, BLKW)],
                    sem_out).wait()

        psp = jnp.full((16,), p, jnp.int32)
        for bi in range(ROUND):
            bsp = jnp.full((16,), bi, jnp.int32)

            def row_body(v8, c, bi=bi, bsp=bsp):
                for u in range(8):
                    vi = v8 * 8 + u
                    vil = jnp.full((16,), vi, jnp.int32)
                    lo = plsc.load_gather(buf, [psp, bsp, dlo, kv, vil])
                    hi = plsc.load_gather(buf, [psp, bsp, dhi, kv, vil])
                    sbase = bi * BLKW + vi * DIM
                    stage[p, pl.ds(sbase, 16)] = lo
                    stage[p, pl.ds(sbase + 16, 16)] = hi
                return c

            lax.fori_loop(0, 16, row_body, 0)

        for bi in range(ROUND):
            cb = jnp.minimum(bstart + r * ROUND + bi, bend - 1)
            pltpu.make_async_copy(
                stage.at[p, pl.ds(bi * BLKW, BLKW)],
                out_flat.at[pl.ds(cb * BLKW, BLKW)],
                sem_out).start()
        return carry

    for cp in in_copies(0, 0):
        cp.start()
    lax.fori_loop(0, NROUND, round_body, 0)

    for _ in range(2):
        for bi in range(ROUND):
            pltpu.make_async_copy(
                stage.at[0, pl.ds(bi * BLKW, BLKW)],
                out_flat.at[pl.ds(bstart * BLKW + bi * BLKW, BLKW)],
                sem_out).wait()


@functools.partial(
    pl.kernel,
    out_type=jax.ShapeDtypeStruct((B, DIM), jnp.float32),
    mesh=_mesh(),
    compiler_params=pltpu.CompilerParams(
        needs_layout_passes=False, use_tc_tiling_on_sc=False),
    scratch_types=[
        pltpu.VMEM((RC,), jnp.int32),
        pltpu.VMEM((RC,), jnp.float32),
        pltpu.VMEM((RC, DIM), jnp.float32),
        pltpu.VMEM((CB, DIM), jnp.float32),
        pltpu.SemaphoreType.DMA,
        pltpu.SemaphoreType.DMA,
        pltpu.SemaphoreType.DMA,
    ],
)
def _bag_kernel(table_hbm, idx_hbm, w_hbm, out_hbm,
                idx_v, w_v, rows_v, out_v, sem_in, sem_rows, sem_out):
    wid = lax.axis_index("s") * NC + lax.axis_index("c")
    lane = lax.broadcasted_iota(jnp.int32, (16,), 0)

    def chunk_body(c, carry):
        bag_base = wid * BAGS_PER_W + c * CB
        row_base = bag_base * L

        cp_i = pltpu.make_async_copy(
            idx_hbm.at[pl.ds(row_base, RC)], idx_v, sem_in)
        cp_w = pltpu.make_async_copy(
            w_hbm.at[pl.ds(row_base, RC)], w_v, sem_in)
        cp_i.start()
        cp_w.start()
        cp_i.wait()
        cp_w.wait()

        def remap_body(i, carry):
            v = idx_v[pl.ds(i * 16, 16)]
            r = (((v >> SHB) << SHB) + ((v & (TSUB - 1)) << 2)
                 + ((v >> SHS) & 3))
            idx_v[pl.ds(i * 16, 16)] = r
            return carry

        lax.fori_loop(0, RC // 16, remap_body, 0)

        g = pltpu.make_async_copy(table_hbm.at[idx_v], rows_v, sem_rows)
        g.start()
        g.wait()

        def bag_body(b, carry):
            r0 = b * L
            acc = [jnp.zeros((16,), jnp.float32) for _ in range(4)]
            for j in range(L):
                r = r0 + j
                w16 = plsc.load_gather(w_v, [jnp.full((16,), r, jnp.int32)])
                lo = rows_v[r, pl.ds(0, 16)]
                hi = rows_v[r, pl.ds(16, 16)]
                k = 2 * (j % 2)
                acc[k] = acc[k] + w16 * lo
                acc[k + 1] = acc[k + 1] + w16 * hi
            out_v[b, pl.ds(0, 16)] = acc[0] + acc[2]
            out_v[b, pl.ds(16, 16)] = acc[1] + acc[3]
            return carry

        lax.fori_loop(0, CB, bag_body, 0)

        cp_o = pltpu.make_async_copy(
            out_v, out_hbm.at[pl.ds(bag_base, CB)], sem_out)
        cp_o.start()
        cp_o.wait()
        return carry

    lax.fori_loop(0, NCHUNK, chunk_body, 0)


def kernel(inputs, offsets, per_sample_weights, table):
    del offsets
    lin = _tc_transpose(table.T)
    return _bag_kernel(lin.reshape(VROWS, DIM), inputs, per_sample_weights)
